# baseline (device time: 94969 ns/iter reference)
import jax
import jax.numpy as jnp
from jax import lax
from jax.experimental import pallas as pl
from jax.experimental.pallas import tpu as pltpu

N_DEV = 32


def kernel(x, W1, W2):
    m, k = x.shape
    n = W2.shape[1]
    H = m // 2
    Q = m // 8
    C = m // 32
    n2 = n // 2

    def body(x_ref, w1_ref, w2_ref, out_ref,
             rsx_comm, rsyA_comm, rsyB_comm, rszA_comm, rszB_comm,
             rsx_send, rsx_recv,
             rsyA_send, rsyA_recv, rsyB_send, rsyB_recv,
             rszA_send, rszA_recv, rszB_send, rszB_recv,
             agzA_send, agzA_recv, agzB_send, agzB_recv,
             agyA_send, agyA_recv, agyB_send, agyB_recv,
             agx_send, agx_recv):
        p = lax.axis_index("i")
        z = p // 8
        q = lax.rem(p, 8)
        y = q // 2
        r = lax.rem(q, 2)
        xc = jnp.where(lax.rem(y, 2) == 0, r, 1 - r)

        x_partner = p + 1 - 2 * r

        def y_ring_id(t):
            rn = jnp.where(lax.rem(t, 2) == 0, xc, 1 - xc)
            return z * 8 + t * 2 + rn

        y_next = y_ring_id(lax.rem(y + 1, 4))
        y_prev = y_ring_id(lax.rem(y + 3, 4))
        z_next = lax.rem(p + 8, N_DEV)
        z_prev = lax.rem(p + 24, N_DEV)

        r_mine = xc * H
        r_other = (1 - xc) * H

        deferred = []

        def copy(src, dst, send, recv, dev):
            return pltpu.make_async_remote_copy(
                src_ref=src, dst_ref=dst, send_sem=send, recv_sem=recv,
                device_id=(dev,), device_id_type=pl.DeviceIdType.MESH,
            )

        barrier_sem = pltpu.get_barrier_semaphore()
        for nbr in [x_partner, y_next, y_prev, z_next, z_prev]:
            pl.semaphore_signal(barrier_sem, inc=1, device_id=(nbr,),
                                device_id_type=pl.DeviceIdType.MESH)

        def mlp_block(r0):
            hh = jnp.dot(x_ref[pl.ds(r0, Q), :], w1_ref[...],
                         preferred_element_type=jnp.float32)
            hh = jnp.maximum(hh, 0.0)
            out_ref[pl.ds(r0, Q), :] = jnp.dot(
                hh, w2_ref[...], preferred_element_type=jnp.float32)

        b0 = y
        bm1 = lax.rem(y + 3, 4)
        bp1 = lax.rem(y + 1, 4)
        bp2 = lax.rem(y + 2, 4)

        def slA(base, b, rows):
            return base.at[pl.ds(b, rows), pl.ds(0, n2)]

        def slB(base, b, rows):
            return base.at[pl.ds(b, rows), pl.ds(n2, n2)]

        piece_block = [b0, b0, bm1, bp1, bp2, bp2, bp1, bm1]
        piece_half = ["A", "B", "A", "B", "A", "B", "A", "B"]
        rsx = [None] * 8

        def start_piece(j):
            b = piece_block[j]
            sl = slA if piece_half[j] == "A" else slB
            rsx[j] = copy(sl(out_ref, r_other + b * Q, Q),
                          sl(rsx_comm, b * Q, Q),
                          rsx_send.at[j], rsx_recv.at[j], x_partner)
            rsx[j].start()
            deferred.append(rsx[j])

        mlp_block(r_other + b0 * Q)
        pl.semaphore_wait(barrier_sem, 5)
        start_piece(0)
        start_piece(1)
        mlp_block(r_other + bm1 * Q)
        start_piece(2)
        mlp_block(r_other + bp1 * Q)
        start_piece(3)
        mlp_block(r_other + bp2 * Q)
        for j in (4, 5, 6, 7):
            start_piece(j)

        def add_piece(j):
            b = piece_block[j]
            sl = slA if piece_half[j] == "A" else slB
            rsx[j].wait_recv()
            dst = sl(out_ref, r_mine + b * Q, Q)
            src = sl(rsx_comm, b * Q, Q)
            dst[...] = dst[...] + src[...]

        rsyA = [
            copy(slA(out_ref, r_mine + lax.rem(y - s + 4, 4) * Q, Q),
                 rsyA_comm.at[s], rsyA_send.at[s], rsyA_recv.at[s], y_next)
            for s in range(3)
        ]
        rsyB = [
            copy(slB(out_ref, r_mine + lax.rem(y + s, 4) * Q, Q),
                 rsyB_comm.at[s], rsyB_send.at[s], rsyB_recv.at[s], y_prev)
            for s in range(3)
        ]

        def ring_add_A(s):
            rsyA[s].wait_recv()
            dst = slA(out_ref, r_mine + lax.rem(y - s - 1 + 4, 4) * Q, Q)
            dst[...] = dst[...] + rsyA_comm[s]

        def ring_add_B(s):
            rsyB[s].wait_recv()
            dst = slB(out_ref, r_mine + lax.rem(y + s + 1, 4) * Q, Q)
            dst[...] = dst[...] + rsyB_comm[s]

        mlp_block(r_mine + b0 * Q)
        add_piece(0)
        add_piece(1)
        rsyA[0].start()
        rsyB[0].start()
        deferred.extend([rsyA[0], rsyB[0]])

        mlp_block(r_mine + bm1 * Q)
        add_piece(2)
        ring_add_A(0)
        rsyA[1].start()
        deferred.append(rsyA[1])

        mlp_block(r_mine + bp1 * Q)
        add_piece(3)
        ring_add_B(0)
        rsyB[1].start()
        deferred.append(rsyB[1])

        mlp_block(r_mine + bp2 * Q)
        add_piece(4)
        ring_add_A(1)
        rsyA[2].start()
        deferred.append(rsyA[2])
        add_piece(5)
        ring_add_B(1)
        rsyB[2].start()
        deferred.append(rsyB[2])

        ring_add_A(2)
        add_piece(6)
        ring_add_B(2)
        add_piece(7)

        baseA = r_mine + lax.rem(y + 1, 4) * Q
        baseB = r_mine + lax.rem(y + 3, 4) * Q

        rszA = [
            copy(slA(out_ref, baseA + lax.rem(z - s + 4, 4) * C, C),
                 rszA_comm.at[s], rszA_send.at[s], rszA_recv.at[s], z_next)
            for s in range(3)
        ]
        rszB = [
            copy(slB(out_ref, baseB + lax.rem(z + s, 4) * C, C),
                 rszB_comm.at[s], rszB_send.at[s], rszB_recv.at[s], z_prev)
            for s in range(3)
        ]
        rszA[0].start()
        rszB[0].start()
        deferred.extend([rszA[0], rszB[0]])
        for s in range(3):
            rszA[s].wait_recv()
            dA = slA(out_ref, baseA + lax.rem(z - s - 1 + 4, 4) * C, C)
            dA[...] = dA[...] + rszA_comm[s]
            if s < 2:
                rszA[s + 1].start()
                deferred.append(rszA[s + 1])
            rszB[s].wait_recv()
            dB = slB(out_ref, baseB + lax.rem(z + s + 1, 4) * C, C)
            dB[...] = dB[...] + rszB_comm[s]
            if s < 2:
                rszB[s + 1].start()
                deferred.append(rszB[s + 1])

        agzA, agzB = [], []
        for t in range(3):
            sa = slA(out_ref, baseA + lax.rem(z + 1 - t + 4, 4) * C, C)
            agzA.append(copy(sa, sa, agzA_send.at[t], agzA_recv.at[t],
                             z_next))
            sb = slB(out_ref, baseB + lax.rem(z - 1 + t + 4, 4) * C, C)
            agzB.append(copy(sb, sb, agzB_send.at[t], agzB_recv.at[t],
                             z_prev))
        agzA[0].start()
        agzB[0].start()
        deferred.extend([agzA[0], agzB[0]])
        for t in range(3):
            agzA[t].wait_recv()
            if t < 2:
                agzA[t + 1].start()
                deferred.append(agzA[t + 1])
            agzB[t].wait_recv()
            if t < 2:
                agzB[t + 1].start()
                deferred.append(agzB[t + 1])

        def agx_piece(i, rows0):
            sl = out_ref.at[pl.ds(rows0, Q), :]
            rdma = copy(sl, sl, agx_send.at[i], agx_recv.at[i], x_partner)
            rdma.start()
            deferred.append(rdma)
            return rdma

        agx = []
        for t in range(3):
            sa = slA(out_ref, r_mine + lax.rem(y + 1 - t + 4, 4) * Q, Q)
            agyA = copy(sa, sa, agyA_send.at[t], agyA_recv.at[t], y_next)
            sb = slB(out_ref, r_mine + lax.rem(y - 1 + t + 4, 4) * Q, Q)
            agyB = copy(sb, sb, agyB_send.at[t], agyB_recv.at[t], y_prev)
            agyA.start()
            agyB.start()
            deferred.extend([agyA, agyB])
            agyA.wait_recv()
            agyB.wait_recv()
            if t == 0:
                agx.append(agx_piece(0, r_mine + b0 * Q))
            elif t == 1:
                agx.append(agx_piece(1, baseA))
                agx.append(agx_piece(2, baseB))
            else:
                agx.append(agx_piece(3, r_mine + bp2 * Q))

        for rdma in agx:
            rdma.wait_recv()
        for rdma in deferred:
            rdma.wait_send()

    return pl.pallas_call(
        body,
        out_shape=jax.ShapeDtypeStruct((m, n), jnp.float32),
        in_specs=[
            pl.BlockSpec(memory_space=pltpu.VMEM),
            pl.BlockSpec(memory_space=pltpu.VMEM),
            pl.BlockSpec(memory_space=pltpu.VMEM),
        ],
        out_specs=pl.BlockSpec(memory_space=pltpu.VMEM),
        scratch_shapes=[
            pltpu.VMEM((H, n), jnp.float32),
            pltpu.VMEM((3, Q, n2), jnp.float32),
            pltpu.VMEM((3, Q, n2), jnp.float32),
            pltpu.VMEM((3, C, n2), jnp.float32),
            pltpu.VMEM((3, C, n2), jnp.float32),
            pltpu.SemaphoreType.DMA((8,)),
            pltpu.SemaphoreType.DMA((8,)),
            pltpu.SemaphoreType.DMA((3,)),
            pltpu.SemaphoreType.DMA((3,)),
            pltpu.SemaphoreType.DMA((3,)),
            pltpu.SemaphoreType.DMA((3,)),
            pltpu.SemaphoreType.DMA((3,)),
            pltpu.SemaphoreType.DMA((3,)),
            pltpu.SemaphoreType.DMA((3,)),
            pltpu.SemaphoreType.DMA((3,)),
            pltpu.SemaphoreType.DMA((3,)),
            pltpu.SemaphoreType.DMA((3,)),
            pltpu.SemaphoreType.DMA((3,)),
            pltpu.SemaphoreType.DMA((3,)),
            pltpu.SemaphoreType.DMA((3,)),
            pltpu.SemaphoreType.DMA((3,)),
            pltpu.SemaphoreType.DMA((3,)),
            pltpu.SemaphoreType.DMA((3,)),
            pltpu.SemaphoreType.DMA((4,)),
            pltpu.SemaphoreType.DMA((4,)),
        ],
        compiler_params=pltpu.CompilerParams(collective_id=0),
    )(x, W1, W2)
